# baseline (speedup 1.0000x reference)
import functools

import jax
import jax.numpy as jnp
from jax import lax
from jax.experimental import pallas as pl
from jax.experimental.pallas import tpu as pltpu
from jax.experimental.pallas import tpu_sc as plsc

_NP = 10000
_NCH = 320000
_ED = 128
_HD = 128

_NC = 2
_NS = 16
_L = 16
_NW = _NC * _NS
_CPW = _NCH // _NW
_WG = 128
_WS = 128
_CHZ = 80

_mesh = plsc.VectorSubcoreMesh(core_axis_name="c", subcore_axis_name="s")


_NKG = _NCH // _WG


def _sc_gather(fx, seg2d):
    @functools.partial(
        pl.kernel,
        out_type=jax.ShapeDtypeStruct((_NCH, _HD // 2), jnp.int32),
        mesh=_mesh,
        cost_estimate=pl.CostEstimate(
            flops=0, bytes_accessed=2 * _NCH * _HD * 2 + _NP * _HD * 2,
            transcendentals=0,
        ),
        scratch_types=[
            pltpu.VMEM((_CHZ, _HD // 2), jnp.int32),
            pltpu.VMEM_SHARED((_NP, _HD // 2), jnp.int32),
            pltpu.VMEM((2, _WG), jnp.int32),
            pltpu.VMEM((2, _WG, _HD // 2), jnp.int32),
            pltpu.SemaphoreType.DMA((2,)),
            pltpu.SemaphoreType.DMA((2,)),
        ],
    )
    def k(fx_hbm, seg_hbm, out_hbm, bounce_v, fx_sh, idx_v, rows_v, isem, osem):
        cid = lax.axis_index("c")
        sid = lax.axis_index("s")
        wid = cid * _NS + sid

        @pl.loop(sid * _CHZ, _NP, step=_CHZ * _NS)
        def _(r0):
            pltpu.sync_copy(fx_hbm.at[pl.ds(r0, _CHZ)], bounce_v)
            pltpu.sync_copy(bounce_v, fx_sh.at[pl.ds(r0, _CHZ)])

        plsc.subcore_barrier()

        nk = (_NKG - wid + _NW - 1) // _NW
        pltpu.async_copy(
            seg_hbm.at[0, pl.ds(wid * _WG, _WG)], idx_v.at[0], isem.at[0]
        )

        @pl.loop(0, nk)
        def _(kk):
            p = lax.rem(kk, 2)
            off = (wid + kk * _NW) * _WG
            pltpu.make_async_copy(
                seg_hbm.at[0, pl.ds(off, _WG)], idx_v.at[p], isem.at[p]
            ).wait()

            @pl.when(kk + 1 < nk)
            def _():
                noff = (wid + (kk + 1) * _NW) * _WG
                pltpu.async_copy(
                    seg_hbm.at[0, pl.ds(noff, _WG)], idx_v.at[1 - p], isem.at[1 - p]
                )

            @pl.when(kk >= 2)
            def _():
                pltpu.make_async_copy(
                    rows_v.at[p], out_hbm.at[pl.ds(0, _WG)], osem.at[p]
                ).wait()

            pltpu.sync_copy(fx_sh.at[idx_v.at[p]], rows_v.at[p])
            pltpu.async_copy(rows_v.at[p], out_hbm.at[pl.ds(off, _WG)], osem.at[p])

        @pl.when(nk >= 2)
        def _():
            pltpu.make_async_copy(
                rows_v.at[0], out_hbm.at[pl.ds(0, _WG)], osem.at[lax.rem(nk, 2)]
            ).wait()

        @pl.when(nk >= 1)
        def _():
            pltpu.make_async_copy(
                rows_v.at[0], out_hbm.at[pl.ds(0, _WG)], osem.at[lax.rem(nk + 1, 2)]
            ).wait()

    return k(fx, seg2d)


def _sc_segsum(vals, seg2d):
    @functools.partial(
        pl.kernel,
        out_type=jax.ShapeDtypeStruct((_NC, _NP, _HD), jnp.float32),
        mesh=_mesh,
        cost_estimate=pl.CostEstimate(
            flops=_NCH * _HD, bytes_accessed=_NCH * _HD * 4 + _NCH * 4,
            transcendentals=0,
        ),
        scratch_types=[
            pltpu.VMEM((_CHZ, _HD), jnp.float32),
            pltpu.VMEM_SHARED((_NP, _HD), jnp.float32),
        ],
    )
    def k(vals_hbm, seg_hbm, out_hbm, rows_v, acc_sh):
        cid = lax.axis_index("c")
        sid = lax.axis_index("s")

        @pl.loop(0, _CHZ)
        def _(r):
            @pl.loop(0, _HD, step=_L)
            def _(col):
                rows_v[r, pl.ds(col, _L)] = jnp.zeros((_L,), jnp.float32)

        @pl.loop(sid * _CHZ, _NP, step=_CHZ * _NS)
        def _(r0):
            pltpu.sync_copy(rows_v, acc_sh.at[pl.ds(r0, _CHZ)])

        plsc.subcore_barrier()

        def body(i_vmem, v_vmem):
            pltpu.sync_copy(v_vmem, acc_sh.at[i_vmem.at[0]], add=True)

        pltpu.emit_pipeline(
            body,
            grid=(_NCH // _WS,),
            in_specs=[
                pl.BlockSpec((1, _WS), lambda i: (0, i)),
                pl.BlockSpec((_WS, _HD), lambda i: (i, 0)),
            ],
            out_specs=[],
            core_axis_name=("c", "s"),
            dimension_semantics=(pltpu.PARALLEL,),
        )(seg_hbm, vals_hbm)

        plsc.subcore_barrier()

        @pl.loop(sid * _CHZ, _NP, step=_CHZ * _NS)
        def _(r0):
            pltpu.sync_copy(acc_sh.at[pl.ds(r0, _CHZ)], rows_v)
            pltpu.sync_copy(rows_v, out_hbm.at[cid, pl.ds(r0, _CHZ)])

    return k(vals, seg2d)


def _tc_fx(inputs, W_f, b_f):
    def body(x_ref, w_ref, b_ref, o_ref):
        o_ref[...] = (
            jnp.dot(x_ref[...], w_ref[...], preferred_element_type=jnp.float32)
            + b_ref[...]
        ).astype(jnp.bfloat16)

    return pl.pallas_call(
        body,
        out_shape=jax.ShapeDtypeStruct((_NP, _HD), jnp.bfloat16),
    )(inputs, W_f, b_f)


_MID_R = 2000


def _tc_mid(prev_h, F, prev_c, U_f):
    def body(h_ref, f_ref, c_ref, u_ref, o_ref):
        fh = jnp.dot(h_ref[...], u_ref[...], preferred_element_type=jnp.float32)
        o_ref[...] = jax.nn.sigmoid(fh + f_ref[...].astype(jnp.float32)) * c_ref[...]

    blk = pl.BlockSpec((_MID_R, _HD), lambda i: (i, 0))
    return pl.pallas_call(
        body,
        grid=(_NCH // _MID_R,),
        in_specs=[blk, blk, blk, pl.BlockSpec((_HD, _HD), lambda i: (0, 0))],
        out_specs=blk,
        out_shape=jax.ShapeDtypeStruct((_NCH, _HD), jnp.float32),
    )(prev_h, F, prev_c, U_f)


_FIN_R = 2000


def _tc_final(inputs, hpart, fpart, W_combined, b_combined):
    def body(x_ref, hp_ref, fp_ref, wc_ref, b_ref, oc_ref, oh_ref):
        ht = hp_ref[0] + hp_ref[1]
        fc_term = fp_ref[0] + fp_ref[1]
        z = (
            jnp.dot(x_ref[...], wc_ref[: _ED], preferred_element_type=jnp.float32)
            + jnp.dot(ht, wc_ref[_ED:], preferred_element_type=jnp.float32)
            + b_ref[...]
        )
        z_i = z[:, :_HD]
        z_o = z[:, _HD : 2 * _HD]
        z_u = z[:, 2 * _HD :]
        c = jax.nn.sigmoid(z_i) * jnp.tanh(z_u) + fc_term
        oc_ref[...] = c
        oh_ref[...] = jax.nn.sigmoid(z_o) * jnp.tanh(c)

    blk = pl.BlockSpec((_FIN_R, _HD), lambda i: (i, 0))
    pblk = pl.BlockSpec((_NC, _FIN_R, _HD), lambda i: (0, i, 0))
    return pl.pallas_call(
        body,
        grid=(_NP // _FIN_R,),
        in_specs=[
            blk,
            pblk,
            pblk,
            pl.BlockSpec((_ED + _HD, 3 * _HD), lambda i: (0, 0)),
            pl.BlockSpec((1, 3 * _HD), lambda i: (0, 0)),
        ],
        out_specs=[blk, blk],
        out_shape=[
            jax.ShapeDtypeStruct((_NP, _HD), jnp.float32),
            jax.ShapeDtypeStruct((_NP, _HD), jnp.float32),
        ],
    )(inputs, hpart, fpart, W_combined, b_combined)


def kernel(inputs, prev_c, prev_h, segment_ids, W_combined, b_combined, W_f, U_f, b_f):
    seg2d = segment_ids.astype(jnp.int32).reshape(1, _NCH)
    fx = _tc_fx(inputs, W_f, b_f)
    fx_packed = lax.bitcast_convert_type(
        fx.reshape(_NP, _HD // 2, 2), jnp.int32
    )
    F_packed = _sc_gather(fx_packed, seg2d)
    F = lax.bitcast_convert_type(F_packed, jnp.bfloat16).reshape(_NCH, _HD)
    hpart = _sc_segsum(prev_h, seg2d)
    fc_mul = _tc_mid(prev_h, F, prev_c, U_f)
    fpart = _sc_segsum(fc_mul, seg2d)
    c, h = _tc_final(inputs, hpart, fpart, W_combined, b_combined)
    return (c, h)


# speedup vs baseline: 2.4317x; 2.4317x over previous
import functools

import jax
import jax.numpy as jnp
from jax import lax
from jax.experimental import pallas as pl
from jax.experimental.pallas import tpu as pltpu
from jax.experimental.pallas import tpu_sc as plsc

_NP = 10000
_NCH = 320000
_ED = 128
_HD = 128

_NC = 2
_NS = 16
_L = 16
_NW = _NC * _NS
_CPW = _NCH // _NW
_WG = 128
_WS = 128
_CHZ = 80

_mesh = plsc.VectorSubcoreMesh(core_axis_name="c", subcore_axis_name="s")


_NKG = _NCH // _WG


def _sc_gather(fx, seg2d):
    @functools.partial(
        pl.kernel,
        out_type=jax.ShapeDtypeStruct((_NCH, _HD), jnp.float32),
        mesh=_mesh,
        cost_estimate=pl.CostEstimate(
            flops=0, bytes_accessed=2 * _NCH * _HD * 4 + _NP * _HD * 4,
            transcendentals=0,
        ),
        scratch_types=[
            pltpu.VMEM((_CHZ, _HD), jnp.float32),
            pltpu.VMEM_SHARED((_NP, _HD), jnp.float32),
            pltpu.VMEM((2, _WG), jnp.int32),
            pltpu.VMEM((2, _WG, _HD), jnp.float32),
            pltpu.SemaphoreType.DMA((2,)),
            pltpu.SemaphoreType.DMA((2,)),
        ],
    )
    def k(fx_hbm, seg_hbm, out_hbm, bounce_v, fx_sh, idx_v, rows_v, isem, osem):
        cid = lax.axis_index("c")
        sid = lax.axis_index("s")
        wid = cid * _NS + sid

        @pl.loop(sid * _CHZ, _NP, step=_CHZ * _NS)
        def _(r0):
            pltpu.sync_copy(fx_hbm.at[pl.ds(r0, _CHZ)], bounce_v)
            pltpu.sync_copy(bounce_v, fx_sh.at[pl.ds(r0, _CHZ)])

        plsc.subcore_barrier()

        nk = (_NKG - wid + _NW - 1) // _NW
        pltpu.async_copy(
            seg_hbm.at[0, pl.ds(wid * _WG, _WG)], idx_v.at[0], isem.at[0]
        )

        @pl.loop(0, nk)
        def _(kk):
            p = lax.rem(kk, 2)
            off = (wid + kk * _NW) * _WG
            pltpu.make_async_copy(
                seg_hbm.at[0, pl.ds(off, _WG)], idx_v.at[p], isem.at[p]
            ).wait()

            @pl.when(kk + 1 < nk)
            def _():
                noff = (wid + (kk + 1) * _NW) * _WG
                pltpu.async_copy(
                    seg_hbm.at[0, pl.ds(noff, _WG)], idx_v.at[1 - p], isem.at[1 - p]
                )

            @pl.when(kk >= 2)
            def _():
                pltpu.make_async_copy(
                    rows_v.at[p], out_hbm.at[pl.ds(0, _WG)], osem.at[p]
                ).wait()

            pltpu.sync_copy(fx_sh.at[idx_v.at[p]], rows_v.at[p])
            pltpu.async_copy(rows_v.at[p], out_hbm.at[pl.ds(off, _WG)], osem.at[p])

        @pl.when(nk >= 2)
        def _():
            pltpu.make_async_copy(
                rows_v.at[0], out_hbm.at[pl.ds(0, _WG)], osem.at[lax.rem(nk, 2)]
            ).wait()

        @pl.when(nk >= 1)
        def _():
            pltpu.make_async_copy(
                rows_v.at[0], out_hbm.at[pl.ds(0, _WG)], osem.at[lax.rem(nk + 1, 2)]
            ).wait()

    return k(fx, seg2d)


def _sc_segsum(vals, seg2d):
    @functools.partial(
        pl.kernel,
        out_type=jax.ShapeDtypeStruct((_NC, _NP, _HD), jnp.float32),
        mesh=_mesh,
        cost_estimate=pl.CostEstimate(
            flops=_NCH * _HD, bytes_accessed=_NCH * _HD * 4 + _NCH * 4,
            transcendentals=0,
        ),
        scratch_types=[
            pltpu.VMEM((_CHZ, _HD), jnp.float32),
            pltpu.VMEM_SHARED((_NP, _HD), jnp.float32),
        ],
    )
    def k(vals_hbm, seg_hbm, out_hbm, rows_v, acc_sh):
        cid = lax.axis_index("c")
        sid = lax.axis_index("s")

        @pl.loop(0, _CHZ)
        def _(r):
            @pl.loop(0, _HD, step=_L)
            def _(col):
                rows_v[r, pl.ds(col, _L)] = jnp.zeros((_L,), jnp.float32)

        @pl.loop(sid * _CHZ, _NP, step=_CHZ * _NS)
        def _(r0):
            pltpu.sync_copy(rows_v, acc_sh.at[pl.ds(r0, _CHZ)])

        plsc.subcore_barrier()

        def body(i_vmem, v_vmem):
            pltpu.sync_copy(v_vmem, acc_sh.at[i_vmem.at[0]], add=True)

        pltpu.emit_pipeline(
            body,
            grid=(_NCH // _WS,),
            in_specs=[
                pl.BlockSpec((1, _WS), lambda i: (0, i)),
                pl.BlockSpec((_WS, _HD), lambda i: (i, 0)),
            ],
            out_specs=[],
            core_axis_name=("c", "s"),
            dimension_semantics=(pltpu.PARALLEL,),
        )(seg_hbm, vals_hbm)

        plsc.subcore_barrier()

        @pl.loop(sid * _CHZ, _NP, step=_CHZ * _NS)
        def _(r0):
            pltpu.sync_copy(acc_sh.at[pl.ds(r0, _CHZ)], out_hbm.at[cid, pl.ds(r0, _CHZ)])

    return k(vals, seg2d)


def _tc_fx(inputs, W_f, b_f):
    def body(x_ref, w_ref, b_ref, o_ref):
        o_ref[...] = (
            jnp.dot(x_ref[...], w_ref[...], preferred_element_type=jnp.float32)
            + b_ref[...]
        )

    return pl.pallas_call(
        body,
        out_shape=jax.ShapeDtypeStruct((_NP, _HD), jnp.float32),
    )(inputs, W_f, b_f)


_MID_R = 2000


def _tc_mid(prev_h, F, prev_c, U_f):
    def body(h_ref, f_ref, c_ref, u_ref, o_ref):
        fh = jnp.dot(h_ref[...], u_ref[...], preferred_element_type=jnp.float32)
        o_ref[...] = jax.nn.sigmoid(fh + f_ref[...]) * c_ref[...]

    blk = pl.BlockSpec((_MID_R, _HD), lambda i: (i, 0))
    return pl.pallas_call(
        body,
        grid=(_NCH // _MID_R,),
        in_specs=[blk, blk, blk, pl.BlockSpec((_HD, _HD), lambda i: (0, 0))],
        out_specs=blk,
        out_shape=jax.ShapeDtypeStruct((_NCH, _HD), jnp.float32),
    )(prev_h, F, prev_c, U_f)


_FIN_R = 2000


def _tc_final(inputs, hpart, fpart, W_combined, b_combined):
    def body(x_ref, hp_ref, fp_ref, wc_ref, b_ref, oc_ref, oh_ref):
        ht = hp_ref[0] + hp_ref[1]
        fc_term = fp_ref[0] + fp_ref[1]
        z = (
            jnp.dot(x_ref[...], wc_ref[: _ED], preferred_element_type=jnp.float32)
            + jnp.dot(ht, wc_ref[_ED:], preferred_element_type=jnp.float32)
            + b_ref[...]
        )
        z_i = z[:, :_HD]
        z_o = z[:, _HD : 2 * _HD]
        z_u = z[:, 2 * _HD :]
        c = jax.nn.sigmoid(z_i) * jnp.tanh(z_u) + fc_term
        oc_ref[...] = c
        oh_ref[...] = jax.nn.sigmoid(z_o) * jnp.tanh(c)

    blk = pl.BlockSpec((_FIN_R, _HD), lambda i: (i, 0))
    pblk = pl.BlockSpec((_NC, _FIN_R, _HD), lambda i: (0, i, 0))
    return pl.pallas_call(
        body,
        grid=(_NP // _FIN_R,),
        in_specs=[
            blk,
            pblk,
            pblk,
            pl.BlockSpec((_ED + _HD, 3 * _HD), lambda i: (0, 0)),
            pl.BlockSpec((1, 3 * _HD), lambda i: (0, 0)),
        ],
        out_specs=[blk, blk],
        out_shape=[
            jax.ShapeDtypeStruct((_NP, _HD), jnp.float32),
            jax.ShapeDtypeStruct((_NP, _HD), jnp.float32),
        ],
    )(inputs, hpart, fpart, W_combined, b_combined)


def kernel(inputs, prev_c, prev_h, segment_ids, W_combined, b_combined, W_f, U_f, b_f):
    seg2d = segment_ids.astype(jnp.int32).reshape(1, _NCH)
    fx = _tc_fx(inputs, W_f, b_f)
    F = _sc_gather(fx, seg2d)
    hpart = _sc_segsum(prev_h, seg2d)
    fc_mul = _tc_mid(prev_h, F, prev_c, U_f)
    fpart = _sc_segsum(fc_mul, seg2d)
    c, h = _tc_final(inputs, hpart, fpart, W_combined, b_combined)
    return (c, h)


# speedup vs baseline: 2.5374x; 1.0434x over previous
import functools

import jax
import jax.numpy as jnp
from jax import lax
from jax.experimental import pallas as pl
from jax.experimental.pallas import tpu as pltpu
from jax.experimental.pallas import tpu_sc as plsc

_NP = 10000
_NCH = 320000
_ED = 128
_HD = 128

_NC = 2
_NS = 16
_L = 16
_NW = _NC * _NS
_CPW = _NCH // _NW
_WG = 128
_WS = 128
_CHZ = 80

_mesh = plsc.VectorSubcoreMesh(core_axis_name="c", subcore_axis_name="s")


_NKG = _NCH // _WG


def _sc_gather(fx, seg2d):
    @functools.partial(
        pl.kernel,
        out_type=jax.ShapeDtypeStruct((_NCH, _HD), jnp.float32),
        mesh=_mesh,
        cost_estimate=pl.CostEstimate(
            flops=0, bytes_accessed=2 * _NCH * _HD * 4 + _NP * _HD * 4,
            transcendentals=0,
        ),
        scratch_types=[
            pltpu.VMEM((_CHZ, _HD), jnp.float32),
            pltpu.VMEM_SHARED((_NP, _HD), jnp.float32),
            pltpu.VMEM((2, _WG), jnp.int32),
            pltpu.VMEM((2, _WG, _HD), jnp.float32),
            pltpu.SemaphoreType.DMA((2,)),
            pltpu.SemaphoreType.DMA((2,)),
        ],
    )
    def k(fx_hbm, seg_hbm, out_hbm, bounce_v, fx_sh, idx_v, rows_v, isem, osem):
        cid = lax.axis_index("c")
        sid = lax.axis_index("s")
        wid = cid * _NS + sid

        @pl.loop(sid * _CHZ, _NP, step=_CHZ * _NS)
        def _(r0):
            pltpu.sync_copy(fx_hbm.at[pl.ds(r0, _CHZ)], bounce_v)
            pltpu.sync_copy(bounce_v, fx_sh.at[pl.ds(r0, _CHZ)])

        plsc.subcore_barrier()

        nk = (_NKG - wid + _NW - 1) // _NW
        pltpu.async_copy(
            seg_hbm.at[0, pl.ds(wid * _WG, _WG)], idx_v.at[0], isem.at[0]
        )

        @pl.loop(0, nk)
        def _(kk):
            p = lax.rem(kk, 2)
            off = (wid + kk * _NW) * _WG
            pltpu.make_async_copy(
                seg_hbm.at[0, pl.ds(off, _WG)], idx_v.at[p], isem.at[p]
            ).wait()

            @pl.when(kk + 1 < nk)
            def _():
                noff = (wid + (kk + 1) * _NW) * _WG
                pltpu.async_copy(
                    seg_hbm.at[0, pl.ds(noff, _WG)], idx_v.at[1 - p], isem.at[1 - p]
                )

            @pl.when(kk >= 2)
            def _():
                pltpu.make_async_copy(
                    rows_v.at[p], out_hbm.at[pl.ds(0, _WG)], osem.at[p]
                ).wait()

            pltpu.sync_copy(fx_sh.at[idx_v.at[p]], rows_v.at[p])
            pltpu.async_copy(rows_v.at[p], out_hbm.at[pl.ds(off, _WG)], osem.at[p])

        @pl.when(nk >= 2)
        def _():
            pltpu.make_async_copy(
                rows_v.at[0], out_hbm.at[pl.ds(0, _WG)], osem.at[lax.rem(nk, 2)]
            ).wait()

        @pl.when(nk >= 1)
        def _():
            pltpu.make_async_copy(
                rows_v.at[0], out_hbm.at[pl.ds(0, _WG)], osem.at[lax.rem(nk + 1, 2)]
            ).wait()

    return k(fx, seg2d)


def _sc_segsum(vals, seg2d):
    @functools.partial(
        pl.kernel,
        out_type=jax.ShapeDtypeStruct((_NC, _NP, _HD), jnp.float32),
        mesh=_mesh,
        cost_estimate=pl.CostEstimate(
            flops=_NCH * _HD, bytes_accessed=_NCH * _HD * 4 + _NCH * 4,
            transcendentals=0,
        ),
        scratch_types=[
            pltpu.VMEM((_CHZ, _HD), jnp.float32),
            pltpu.VMEM_SHARED((_NP, _HD), jnp.float32),
        ],
    )
    def k(vals_hbm, seg_hbm, out_hbm, rows_v, acc_sh):
        cid = lax.axis_index("c")
        sid = lax.axis_index("s")

        @pl.loop(0, _CHZ)
        def _(r):
            @pl.loop(0, _HD, step=_L)
            def _(col):
                rows_v[r, pl.ds(col, _L)] = jnp.zeros((_L,), jnp.float32)

        @pl.loop(sid * _CHZ, _NP, step=_CHZ * _NS)
        def _(r0):
            pltpu.sync_copy(rows_v, acc_sh.at[pl.ds(r0, _CHZ)])

        plsc.subcore_barrier()

        def body(i_vmem, v_vmem):
            pltpu.sync_copy(v_vmem, acc_sh.at[i_vmem.at[0]], add=True)

        pltpu.emit_pipeline(
            body,
            grid=(_NCH // _WS,),
            in_specs=[
                pl.BlockSpec((1, _WS), lambda i: (0, i)),
                pl.BlockSpec((_WS, _HD), lambda i: (i, 0)),
            ],
            out_specs=[],
            core_axis_name=("c", "s"),
            dimension_semantics=(pltpu.PARALLEL,),
        )(seg_hbm, vals_hbm)

        plsc.subcore_barrier()

        @pl.loop(sid * _CHZ, _NP, step=_CHZ * _NS)
        def _(r0):
            pltpu.sync_copy(acc_sh.at[pl.ds(r0, _CHZ)], out_hbm.at[cid, pl.ds(r0, _CHZ)])

    return k(vals, seg2d)


def _tc_fx(inputs, W_f, b_f):
    def body(x_ref, w_ref, b_ref, o_ref):
        o_ref[...] = (
            jnp.dot(x_ref[...], w_ref[...], preferred_element_type=jnp.float32)
            + b_ref[...]
        )

    return pl.pallas_call(
        body,
        out_shape=jax.ShapeDtypeStruct((_NP, _HD), jnp.float32),
    )(inputs, W_f, b_f)


_MID_R = 4000


def _tc_mid(prev_h, F, prev_c, U_f):
    def body(h_ref, f_ref, c_ref, u_ref, o_ref):
        fh = jnp.dot(h_ref[...], u_ref[...], preferred_element_type=jnp.float32)
        o_ref[...] = jax.nn.sigmoid(fh + f_ref[...]) * c_ref[...]

    blk = pl.BlockSpec((_MID_R, _HD), lambda i: (i, 0))
    return pl.pallas_call(
        body,
        grid=(_NCH // _MID_R,),
        in_specs=[blk, blk, blk, pl.BlockSpec((_HD, _HD), lambda i: (0, 0))],
        out_specs=blk,
        out_shape=jax.ShapeDtypeStruct((_NCH, _HD), jnp.float32),
    )(prev_h, F, prev_c, U_f)


_FIN_R = 2000


def _tc_final(inputs, hpart, fpart, W_combined, b_combined):
    def body(x_ref, hp_ref, fp_ref, wc_ref, b_ref, oc_ref, oh_ref):
        ht = hp_ref[0] + hp_ref[1]
        fc_term = fp_ref[0] + fp_ref[1]
        z = (
            jnp.dot(x_ref[...], wc_ref[: _ED], preferred_element_type=jnp.float32)
            + jnp.dot(ht, wc_ref[_ED:], preferred_element_type=jnp.float32)
            + b_ref[...]
        )
        z_i = z[:, :_HD]
        z_o = z[:, _HD : 2 * _HD]
        z_u = z[:, 2 * _HD :]
        c = jax.nn.sigmoid(z_i) * jnp.tanh(z_u) + fc_term
        oc_ref[...] = c
        oh_ref[...] = jax.nn.sigmoid(z_o) * jnp.tanh(c)

    blk = pl.BlockSpec((_FIN_R, _HD), lambda i: (i, 0))
    pblk = pl.BlockSpec((_NC, _FIN_R, _HD), lambda i: (0, i, 0))
    return pl.pallas_call(
        body,
        grid=(_NP // _FIN_R,),
        in_specs=[
            blk,
            pblk,
            pblk,
            pl.BlockSpec((_ED + _HD, 3 * _HD), lambda i: (0, 0)),
            pl.BlockSpec((1, 3 * _HD), lambda i: (0, 0)),
        ],
        out_specs=[blk, blk],
        out_shape=[
            jax.ShapeDtypeStruct((_NP, _HD), jnp.float32),
            jax.ShapeDtypeStruct((_NP, _HD), jnp.float32),
        ],
    )(inputs, hpart, fpart, W_combined, b_combined)


def kernel(inputs, prev_c, prev_h, segment_ids, W_combined, b_combined, W_f, U_f, b_f):
    seg2d = segment_ids.astype(jnp.int32).reshape(1, _NCH)
    fx = _tc_fx(inputs, W_f, b_f)
    F = _sc_gather(fx, seg2d)
    hpart = _sc_segsum(prev_h, seg2d)
    fc_mul = _tc_mid(prev_h, F, prev_c, U_f)
    fpart = _sc_segsum(fc_mul, seg2d)
    c, h = _tc_final(inputs, hpart, fpart, W_combined, b_combined)
    return (c, h)


# speedup vs baseline: 2.5457x; 1.0033x over previous
import functools

import jax
import jax.numpy as jnp
from jax import lax
from jax.experimental import pallas as pl
from jax.experimental.pallas import tpu as pltpu
from jax.experimental.pallas import tpu_sc as plsc

_NP = 10000
_NCH = 320000
_ED = 128
_HD = 128

_NC = 2
_NS = 16
_L = 16
_NW = _NC * _NS
_CPW = _NCH // _NW
_WG = 128
_WS = 128
_CHZ = 80

_mesh = plsc.VectorSubcoreMesh(core_axis_name="c", subcore_axis_name="s")


_NKG = _NCH // _WG


def _sc_gather(fx, seg2d):
    @functools.partial(
        pl.kernel,
        out_type=jax.ShapeDtypeStruct((_NCH, _HD), jnp.float32),
        mesh=_mesh,
        cost_estimate=pl.CostEstimate(
            flops=0, bytes_accessed=2 * _NCH * _HD * 4 + _NP * _HD * 4,
            transcendentals=0,
        ),
        scratch_types=[
            pltpu.VMEM((_CHZ, _HD), jnp.float32),
            pltpu.VMEM_SHARED((_NP, _HD), jnp.float32),
            pltpu.VMEM((2, _WG), jnp.int32),
            pltpu.VMEM((2, _WG, _HD), jnp.float32),
            pltpu.SemaphoreType.DMA((2,)),
            pltpu.SemaphoreType.DMA((2,)),
        ],
    )
    def k(fx_hbm, seg_hbm, out_hbm, bounce_v, fx_sh, idx_v, rows_v, isem, osem):
        cid = lax.axis_index("c")
        sid = lax.axis_index("s")
        wid = cid * _NS + sid

        @pl.loop(sid * _CHZ, _NP, step=_CHZ * _NS)
        def _(r0):
            pltpu.sync_copy(fx_hbm.at[pl.ds(r0, _CHZ)], bounce_v)
            pltpu.sync_copy(bounce_v, fx_sh.at[pl.ds(r0, _CHZ)])

        plsc.subcore_barrier()

        nk = (_NKG - wid + _NW - 1) // _NW
        pltpu.async_copy(
            seg_hbm.at[0, pl.ds(wid * _WG, _WG)], idx_v.at[0], isem.at[0]
        )

        @pl.loop(0, nk)
        def _(kk):
            p = lax.rem(kk, 2)
            off = (wid + kk * _NW) * _WG
            pltpu.make_async_copy(
                seg_hbm.at[0, pl.ds(off, _WG)], idx_v.at[p], isem.at[p]
            ).wait()

            @pl.when(kk + 1 < nk)
            def _():
                noff = (wid + (kk + 1) * _NW) * _WG
                pltpu.async_copy(
                    seg_hbm.at[0, pl.ds(noff, _WG)], idx_v.at[1 - p], isem.at[1 - p]
                )

            @pl.when(kk >= 2)
            def _():
                pltpu.make_async_copy(
                    rows_v.at[p], out_hbm.at[pl.ds(0, _WG)], osem.at[p]
                ).wait()

            pltpu.sync_copy(fx_sh.at[idx_v.at[p]], rows_v.at[p])
            pltpu.async_copy(rows_v.at[p], out_hbm.at[pl.ds(off, _WG)], osem.at[p])

        @pl.when(nk >= 2)
        def _():
            pltpu.make_async_copy(
                rows_v.at[0], out_hbm.at[pl.ds(0, _WG)], osem.at[lax.rem(nk, 2)]
            ).wait()

        @pl.when(nk >= 1)
        def _():
            pltpu.make_async_copy(
                rows_v.at[0], out_hbm.at[pl.ds(0, _WG)], osem.at[lax.rem(nk + 1, 2)]
            ).wait()

    return k(fx, seg2d)


def _sc_segsum(vals, seg2d):
    @functools.partial(
        pl.kernel,
        out_type=jax.ShapeDtypeStruct((_NC, _NP, _HD), jnp.float32),
        mesh=_mesh,
        cost_estimate=pl.CostEstimate(
            flops=_NCH * _HD, bytes_accessed=_NCH * _HD * 4 + _NCH * 4,
            transcendentals=0,
        ),
        scratch_types=[
            pltpu.VMEM((_CHZ, _HD), jnp.float32),
            pltpu.VMEM_SHARED((_NP, _HD), jnp.float32),
        ],
    )
    def k(vals_hbm, seg_hbm, out_hbm, rows_v, acc_sh):
        cid = lax.axis_index("c")
        sid = lax.axis_index("s")

        @pl.loop(0, _CHZ)
        def _(r):
            @pl.loop(0, _HD, step=_L)
            def _(col):
                rows_v[r, pl.ds(col, _L)] = jnp.zeros((_L,), jnp.float32)

        @pl.loop(sid * _CHZ, _NP, step=_CHZ * _NS)
        def _(r0):
            pltpu.sync_copy(rows_v, acc_sh.at[pl.ds(r0, _CHZ)])

        plsc.subcore_barrier()

        def body(i_vmem, v_vmem):
            pltpu.sync_copy(v_vmem, acc_sh.at[i_vmem.at[0]], add=True)

        pltpu.emit_pipeline(
            body,
            grid=(_NCH // _WS,),
            in_specs=[
                pl.BlockSpec((1, _WS), lambda i: (0, i)),
                pl.BlockSpec((_WS, _HD), lambda i: (i, 0)),
            ],
            out_specs=[],
            core_axis_name=("c", "s"),
            dimension_semantics=(pltpu.PARALLEL,),
        )(seg_hbm, vals_hbm)

        plsc.subcore_barrier()

        @pl.loop(sid * _CHZ, _NP, step=_CHZ * _NS)
        def _(r0):
            pltpu.sync_copy(acc_sh.at[pl.ds(r0, _CHZ)], out_hbm.at[cid, pl.ds(r0, _CHZ)])

    return k(vals, seg2d)


def _tc_fx(inputs, W_f, b_f):
    def body(x_ref, w_ref, b_ref, o_ref):
        o_ref[...] = (
            jnp.dot(x_ref[...], w_ref[...], preferred_element_type=jnp.float32)
            + b_ref[...]
        )

    return pl.pallas_call(
        body,
        out_shape=jax.ShapeDtypeStruct((_NP, _HD), jnp.float32),
    )(inputs, W_f, b_f)


_MID_R = 8000


def _tc_mid(prev_h, F, prev_c, U_f):
    def body(h_ref, f_ref, c_ref, u_ref, o_ref):
        fh = jnp.dot(h_ref[...], u_ref[...], preferred_element_type=jnp.float32)
        o_ref[...] = jax.nn.sigmoid(fh + f_ref[...]) * c_ref[...]

    blk = pl.BlockSpec((_MID_R, _HD), lambda i: (i, 0))
    return pl.pallas_call(
        body,
        grid=(_NCH // _MID_R,),
        in_specs=[blk, blk, blk, pl.BlockSpec((_HD, _HD), lambda i: (0, 0))],
        out_specs=blk,
        out_shape=jax.ShapeDtypeStruct((_NCH, _HD), jnp.float32),
    )(prev_h, F, prev_c, U_f)


_FIN_R = 2000


def _tc_final(inputs, hpart, fpart, W_combined, b_combined):
    def body(x_ref, hp_ref, fp_ref, wc_ref, b_ref, oc_ref, oh_ref):
        ht = hp_ref[0] + hp_ref[1]
        fc_term = fp_ref[0] + fp_ref[1]
        z = (
            jnp.dot(x_ref[...], wc_ref[: _ED], preferred_element_type=jnp.float32)
            + jnp.dot(ht, wc_ref[_ED:], preferred_element_type=jnp.float32)
            + b_ref[...]
        )
        z_i = z[:, :_HD]
        z_o = z[:, _HD : 2 * _HD]
        z_u = z[:, 2 * _HD :]
        c = jax.nn.sigmoid(z_i) * jnp.tanh(z_u) + fc_term
        oc_ref[...] = c
        oh_ref[...] = jax.nn.sigmoid(z_o) * jnp.tanh(c)

    blk = pl.BlockSpec((_FIN_R, _HD), lambda i: (i, 0))
    pblk = pl.BlockSpec((_NC, _FIN_R, _HD), lambda i: (0, i, 0))
    return pl.pallas_call(
        body,
        grid=(_NP // _FIN_R,),
        in_specs=[
            blk,
            pblk,
            pblk,
            pl.BlockSpec((_ED + _HD, 3 * _HD), lambda i: (0, 0)),
            pl.BlockSpec((1, 3 * _HD), lambda i: (0, 0)),
        ],
        out_specs=[blk, blk],
        out_shape=[
            jax.ShapeDtypeStruct((_NP, _HD), jnp.float32),
            jax.ShapeDtypeStruct((_NP, _HD), jnp.float32),
        ],
    )(inputs, hpart, fpart, W_combined, b_combined)


def kernel(inputs, prev_c, prev_h, segment_ids, W_combined, b_combined, W_f, U_f, b_f):
    seg2d = segment_ids.astype(jnp.int32).reshape(1, _NCH)
    fx = _tc_fx(inputs, W_f, b_f)
    F = _sc_gather(fx, seg2d)
    hpart = _sc_segsum(prev_h, seg2d)
    fc_mul = _tc_mid(prev_h, F, prev_c, U_f)
    fpart = _sc_segsum(fc_mul, seg2d)
    c, h = _tc_final(inputs, hpart, fpart, W_combined, b_combined)
    return (c, h)
